# baseline (speedup 1.0000x reference)
import functools

import jax
import jax.numpy as jnp
from jax import lax
from jax.experimental import pallas as pl
from jax.experimental.pallas import tpu as pltpu
from jax.experimental.pallas import tpu_sc as plsc

_VOCAB = 1000000
_DIM = 64
_B = 16384
_L = 50

_NUM_CORES = 2
_NUM_SUBCORES = 16
_NW = _NUM_CORES * _NUM_SUBCORES
_BPW = _B // _NW
_NBUF = 3


def _make_gather_kernel():
  mesh = plsc.VectorSubcoreMesh(core_axis_name="c", subcore_axis_name="s")

  @functools.partial(
      pl.kernel,
      mesh=mesh,
      out_type=jax.ShapeDtypeStruct((_L, _B, _DIM), jnp.float32),
      scratch_types=[
          pltpu.VMEM((_L, _BPW), jnp.int32),
          pltpu.VMEM((_BPW, _DIM), jnp.float32),
          pltpu.VMEM((_BPW, _DIM), jnp.float32),
          pltpu.VMEM((_BPW, _DIM), jnp.float32),
          pltpu.SemaphoreType.DMA,
          pltpu.SemaphoreType.DMA,
          pltpu.SemaphoreType.DMA,
          pltpu.SemaphoreType.DMA,
          pltpu.SemaphoreType.DMA,
          pltpu.SemaphoreType.DMA,
      ],
      compiler_params=pltpu.CompilerParams(use_tc_tiling_on_sc=False),
  )
  def gather_kernel(idx_hbm, table_hbm, out_hbm, idx_v, rows0, rows1, rows2,
                    sem_g0, sem_g1, sem_g2, sem_o0, sem_o1, sem_o2):
    wid = lax.axis_index("s") * _NUM_CORES + lax.axis_index("c")
    base_b = wid * _BPW
    rows = (rows0, rows1, rows2)
    sem_g = (sem_g0, sem_g1, sem_g2)
    sem_o = (sem_o0, sem_o1, sem_o2)

    def start_gather(l, b):
      pltpu.async_copy(table_hbm.at[idx_v.at[l]], rows[b], sem_g[b])

    def wait_gather(l, b):
      pltpu.make_async_copy(table_hbm.at[idx_v.at[l]], rows[b],
                            sem_g[b]).wait()

    def start_out(l, b):
      pltpu.async_copy(rows[b], out_hbm.at[l, pl.ds(base_b, _BPW)], sem_o[b])

    def wait_out(l, b):
      pltpu.make_async_copy(rows[b], out_hbm.at[l, pl.ds(base_b, _BPW)],
                            sem_o[b]).wait()

    pltpu.sync_copy(idx_hbm.at[pl.ds(0, _L), pl.ds(base_b, _BPW)], idx_v)

    pref = _NBUF - 1
    for l in range(pref):
      start_gather(l, l)
    for l in range(_L):
      b = l % _NBUF
      if l + pref < _L:
        if l >= 1:
          wait_out(l - 1, (l + pref) % _NBUF)
        start_gather(l + pref, (l + pref) % _NBUF)
      wait_gather(l, b)
      start_out(l, b)
    for l in range(_L - _NBUF, _L):
      wait_out(l, l % _NBUF)

  return gather_kernel


_gather = _make_gather_kernel()


@jax.jit
def kernel(input, table):
  idx_t = input.T.astype(jnp.int32)
  k = _gather(idx_t, table)
  return k.transpose(1, 0, 2)
